# Initial kernel scaffold; baseline (speedup 1.0000x reference)
import jax
import jax.numpy as jnp
from jax.experimental import pallas as pl


def kernel(x, edge_index, edge_attr, params):
    raise NotImplementedError("write your pallas kernel here")


# speedup vs baseline: 3.3578x; 3.3578x over previous
import functools

import jax
import jax.numpy as jnp
from jax import lax
from jax.experimental import pallas as pl
from jax.experimental.pallas import tpu as pltpu
from jax.experimental.pallas import tpu_sc as plsc

_N = 10000
_E = 320000
_D = 128
_DH = _D // 2
_EDGE_DIM = 16
_HID = 32
_L = 10

_NC = 2
_NS = 16
_NW = _NC * _NS
_CHUNK = 128
_EPW = 10112
_NCHUNK = _EPW // _CHUNK
_EPAD = _NW * _EPW
_EPT = _EPAD // _NS
_NCHUNK2 = _EPT // _CHUNK
_BE = 4096
_NP = 10240
_RPS = _NP // _NS
_NH = 5120
_NHD = _NH + 8
_RPH = _NH // _NS


def _edge_mlp_body(ea_ref, w1_ref, b1_ref, w2_ref, b2_ref, wr_ref, wt_ref):
    i = pl.program_id(0)
    h = jnp.dot(ea_ref[...], w1_ref[...], preferred_element_type=jnp.float32)
    h = jnp.maximum(h + b1_ref[...], 0.0)
    z = jnp.dot(h, w2_ref[...], preferred_element_type=jnp.float32) + b2_ref[...]
    w = jax.nn.sigmoid(z)
    eid = i * _BE + lax.broadcasted_iota(jnp.int32, (_BE, _EDGE_DIM), 0)
    w = jnp.where(eid < _E, w, 0.0)
    wr_ref[...] = w
    wt_ref[...] = w.T


_edge_mlp = pl.pallas_call(
    _edge_mlp_body,
    grid=(_EPAD // _BE,),
    in_specs=[
        pl.BlockSpec((_BE, _EDGE_DIM), lambda i: (i, 0)),
        pl.BlockSpec((_EDGE_DIM, _L * _HID), lambda i: (0, 0)),
        pl.BlockSpec((1, _L * _HID), lambda i: (0, 0)),
        pl.BlockSpec((_L * _HID, 16), lambda i: (0, 0)),
        pl.BlockSpec((1, 16), lambda i: (0, 0)),
    ],
    out_specs=[
        pl.BlockSpec((_BE, 16), lambda i: (i, 0)),
        pl.BlockSpec((16, _BE), lambda i: (0, i)),
    ],
    out_shape=[
        jax.ShapeDtypeStruct((_EPAD, 16), jnp.float32),
        jax.ShapeDtypeStruct((16, _EPAD), jnp.float32),
    ],
)


def _dinv_body(parts_ref, dinv_ref):
    deg = 1.0 + parts_ref[0] + parts_ref[1]
    dinv_ref[...] = lax.rsqrt(deg)


_dinv_call = pl.pallas_call(
    _dinv_body,
    out_shape=jax.ShapeDtypeStruct((_NP, 16), jnp.float32),
)


def _mm_scale_body(h_ref, wt_ref, dinv_ref, y_ref):
    xp = jnp.dot(h_ref[...], wt_ref[...], preferred_element_type=jnp.float32)
    y_ref[...] = dinv_ref[...] * xp


_mm_scale = pl.pallas_call(
    _mm_scale_body,
    out_shape=jax.ShapeDtypeStruct((_N, _D), jnp.float32),
)


def _mid_body(acc_ref, y_ref, dinv_ref, dinvn_ref, b_ref, a_ref,
              nw_ref, nb_ref, ms_ref, wtn_ref, ynext_ref, out_ref):
    acc = jnp.concatenate([acc_ref[0, :_N], acc_ref[1, :_N]], axis=1)
    out = dinv_ref[...] * (acc + y_ref[...]) + b_ref[...]
    a = a_ref[0, 0]
    t = jnp.where(out >= 0.0, out, a * out)
    mean = jnp.mean(t, axis=0, keepdims=True)
    o = t - mean * ms_ref[...]
    var = jnp.mean(o * o, axis=0, keepdims=True)
    g = nw_ref[...] * o * lax.rsqrt(var + 1e-5) + nb_ref[...]
    xp = jnp.dot(g, wtn_ref[...], preferred_element_type=jnp.float32)
    ynext_ref[...] = dinvn_ref[...] * xp
    out_ref[...] = out


_mid_call = pl.pallas_call(
    _mid_body,
    out_shape=[
        jax.ShapeDtypeStruct((_N, _D), jnp.float32),
        jax.ShapeDtypeStruct((_N, _D), jnp.float32),
    ],
)


_MESH = plsc.VectorSubcoreMesh(core_axis_name="c", subcore_axis_name="s")


def _zero_fill(ref, rows, cols16):
    z = jnp.zeros((16,), jnp.float32)

    def body(r, carry):
        for j in range(cols16):
            ref[r, pl.ds(16 * j, 16)] = z
        return carry

    lax.fori_loop(0, rows, body, 0)


@functools.partial(
    pl.kernel,
    out_type=jax.ShapeDtypeStruct((_NC, _NP, 16), jnp.float32),
    mesh=_MESH,
    scratch_types=[
        pltpu.VMEM((_CHUNK,), jnp.int32),
        pltpu.VMEM((_CHUNK, 16), jnp.float32),
        pltpu.VMEM((_RPS, 16), jnp.float32),
        pltpu.VMEM_SHARED((_NP, 16), jnp.float32),
        pltpu.SemaphoreType.DMA,
    ],
)
def _deg_kernel(col_hbm, wr_hbm, out_hbm, idx_v, w_v, buf_v, acc_sh, sem):
    cid = lax.axis_index("c")
    sid = lax.axis_index("s")
    wid = cid * _NS + sid

    _zero_fill(w_v, _CHUNK, 1)

    def _ramp(base):
        for g in range(_CHUNK // 16):
            idx_v[pl.ds(16 * g, 16)] = lax.iota(jnp.int32, 16) + (base + 16 * g)

    def zchunk(c, carry):
        _ramp(sid * _RPS + c * _CHUNK)
        pltpu.sync_copy(w_v, acc_sh.at[idx_v])
        return carry

    lax.fori_loop(0, _RPS // _CHUNK, zchunk, 0)
    plsc.subcore_barrier()

    def chunk(k, carry):
        base = wid * _EPW + k * _CHUNK
        pltpu.sync_copy(col_hbm.at[pl.ds(base, _CHUNK)], idx_v)
        pltpu.sync_copy(wr_hbm.at[pl.ds(base, _CHUNK)], w_v)
        pltpu.sync_copy(w_v, acc_sh.at[idx_v], add=True)
        return carry

    lax.fori_loop(0, _NCHUNK, chunk, 0)
    plsc.subcore_barrier()

    def cchunk(c, carry):
        base = sid * _RPS + c * _CHUNK
        _ramp(base)
        pltpu.sync_copy(acc_sh.at[idx_v], w_v)
        pltpu.sync_copy(w_v, out_hbm.at[cid, pl.ds(base, _CHUNK)])
        return carry

    lax.fori_loop(0, _RPS // _CHUNK, cchunk, 0)


@functools.partial(
    pl.kernel,
    out_type=jax.ShapeDtypeStruct((_NC, _NP, _DH), jnp.float32),
    mesh=_MESH,
    scratch_types=[
        pltpu.VMEM((_CHUNK,), jnp.int32),
        pltpu.VMEM((_CHUNK,), jnp.int32),
        pltpu.VMEM((_CHUNK,), jnp.float32),
        pltpu.VMEM((_CHUNK, _D), jnp.float32),
        pltpu.VMEM((_CHUNK, _DH), jnp.float32),
        pltpu.VMEM((64,), jnp.int32),
        pltpu.VMEM((64, _DH), jnp.float32),
        pltpu.VMEM_SHARED((_NHD, _DH), jnp.float32),
        pltpu.SemaphoreType.DMA,
    ],
)
def _agg_kernel(y_hbm, row_hbm, col_hbm, wl_hbm, out_hbm,
                idx_r, idx_c, w_v, msg_v, half_v, zidx_v, zbuf_v, acc_sh, sem):
    cid = lax.axis_index("c")
    sid = lax.axis_index("s")
    foff = cid * _DH

    _zero_fill(zbuf_v, 64, _DH // 16)

    def _ramp64(base):
        for g in range(4):
            zidx_v[pl.ds(16 * g, 16)] = lax.iota(jnp.int32, 16) + (base + 16 * g)

    for p in range(2):
        def zchunk(c, carry):
            _ramp64(sid * _RPH + c * 64)
            pltpu.sync_copy(zbuf_v, acc_sh.at[zidx_v])
            return carry

        lax.fori_loop(0, _RPH // 64, zchunk, 0)
        plsc.subcore_barrier()

        def chunk(k, carry):
            base = sid * _EPT + k * _CHUNK
            pltpu.sync_copy(row_hbm.at[pl.ds(base, _CHUNK)], idx_r)
            gather = pltpu.async_copy(y_hbm.at[idx_r], msg_v, sem)
            pltpu.sync_copy(wl_hbm.at[pl.ds(base, _CHUNK)], w_v)
            pltpu.sync_copy(col_hbm.at[pl.ds(base, _CHUNK)], idx_c)
            for g in range(_CHUNK // 16):
                sl = pl.ds(16 * g, 16)
                lv = idx_c[sl] - (p * _NH)
                ok = (lv >= 0) & (lv < _NH)
                idx_c[sl] = jnp.where(ok, lv, _NH)
            gather.wait()

            def scale(g, c2):
                wvec = w_v[pl.ds(16 * g, 16)]
                for t in range(16):
                    wv = wvec[t]
                    i = 16 * g + t
                    for j in range(_DH // 16):
                        v = msg_v[i, pl.ds(foff + 16 * j, 16)]
                        half_v[i, pl.ds(16 * j, 16)] = v * wv
                return c2

            lax.fori_loop(0, _CHUNK // 16, scale, 0)
            pltpu.sync_copy(half_v, acc_sh.at[idx_c], add=True)
            return carry

        lax.fori_loop(0, _NCHUNK2, chunk, 0)
        plsc.subcore_barrier()

        def cchunk(c, carry):
            base = sid * _RPH + c * 64
            _ramp64(base)
            pltpu.sync_copy(acc_sh.at[zidx_v], zbuf_v)
            pltpu.sync_copy(zbuf_v, out_hbm.at[cid, pl.ds(p * _NH + base, 64)])
            return carry

        lax.fori_loop(0, _RPH // 64, cchunk, 0)
        _zero_fill(zbuf_v, 64, _DH // 16)
        plsc.subcore_barrier()


def kernel(x, edge_index, edge_attr, params):
    convs = params["convs"]
    row = edge_index[0]
    col = edge_index[1]
    pad = _EPAD - _E
    row_p = jnp.pad(row, (0, pad))
    col_p = jnp.pad(col, (0, pad))
    ea_p = jnp.pad(edge_attr, ((0, pad), (0, 0)))

    w1cat = jnp.concatenate([c["mW1"].T for c in convs], axis=1)
    b1cat = jnp.concatenate([c["mb1"] for c in convs])[None, :]
    w2blk = jnp.zeros((_L * _HID, 16), jnp.float32)
    b2cat = jnp.zeros((16,), jnp.float32)
    for l in range(_L):
        w2blk = w2blk.at[_HID * l:_HID * (l + 1), l].set(convs[l]["mW2"][0])
        b2cat = b2cat.at[l].set(convs[l]["mb2"][0])
    b2cat = b2cat[None, :]

    w_rows, w_t = _edge_mlp(ea_p, w1cat, b1cat, w2blk, b2cat)

    deg_parts = _deg_kernel(col_p, w_rows)
    dinv_all = _dinv_call(deg_parts)[:_N]

    wts = [c["W"].T for c in convs]
    dinvs = [lax.slice(dinv_all, (0, l), (_N, l + 1)) for l in range(_L)]

    y = _mm_scale(x, wts[0], dinvs[0])

    nrm = params["norms"]
    acts = params["acts"]
    xs = {
        "wl": w_t[:_L],
        "dinv": jnp.stack(dinvs),
        "dinvn": jnp.stack(dinvs[1:] + [dinvs[-1]]),
        "b": jnp.stack([c["b"][None, :] for c in convs]),
        "a": jnp.stack([a.reshape(1, 1) for a in acts] + [acts[0].reshape(1, 1)]),
        "nw": jnp.stack([n["weight"][None, :] for n in nrm] + [nrm[0]["weight"][None, :]]),
        "nb": jnp.stack([n["bias"][None, :] for n in nrm] + [nrm[0]["bias"][None, :]]),
        "ms": jnp.stack([n["mean_scale"][None, :] for n in nrm] + [nrm[0]["mean_scale"][None, :]]),
        "wtn": jnp.stack(wts[1:] + [wts[-1]]),
    }

    def step(carry, s):
        y_c, _ = carry
        acc = _agg_kernel(y_c, row_p, col_p, s["wl"])
        y_n, out = _mid_call(acc, y_c, s["dinv"], s["dinvn"], s["b"], s["a"],
                             s["nw"], s["nb"], s["ms"], s["wtn"])
        return (y_n, out), None

    (_, out), _ = lax.scan(step, (y, jnp.zeros((_N, _D), jnp.float32)), xs)
    return out


# speedup vs baseline: 3.7131x; 1.1058x over previous
import functools

import jax
import jax.numpy as jnp
from jax import lax
from jax.experimental import pallas as pl
from jax.experimental.pallas import tpu as pltpu
from jax.experimental.pallas import tpu_sc as plsc

_N = 10000
_E = 320000
_D = 128
_DH = _D // 2
_EDGE_DIM = 16
_HID = 32
_L = 10

_NC = 2
_NS = 16
_NW = _NC * _NS
_CHUNK = 128
_EPW = 10112
_NCHUNK = _EPW // _CHUNK
_EPAD = _NW * _EPW
_EPT = _EPAD // _NS
_NCHUNK2 = _EPT // _CHUNK
_BE = 4096
_NP = 10240
_RPS = _NP // _NS
_NH = 5120
_NHD = _NH + 8
_RPH = _NH // _NS


def _edge_mlp_body(ea_ref, w1_ref, b1_ref, w2_ref, b2_ref, wr_ref, wt_ref):
    i = pl.program_id(0)
    h = jnp.dot(ea_ref[...], w1_ref[...], preferred_element_type=jnp.float32)
    h = jnp.maximum(h + b1_ref[...], 0.0)
    z = jnp.dot(h, w2_ref[...], preferred_element_type=jnp.float32) + b2_ref[...]
    w = jax.nn.sigmoid(z)
    eid = i * _BE + lax.broadcasted_iota(jnp.int32, (_BE, _EDGE_DIM), 0)
    w = jnp.where(eid < _E, w, 0.0)
    wr_ref[...] = w
    wt_ref[...] = w.T


_edge_mlp = pl.pallas_call(
    _edge_mlp_body,
    grid=(_EPAD // _BE,),
    in_specs=[
        pl.BlockSpec((_BE, _EDGE_DIM), lambda i: (i, 0)),
        pl.BlockSpec((_EDGE_DIM, _L * _HID), lambda i: (0, 0)),
        pl.BlockSpec((1, _L * _HID), lambda i: (0, 0)),
        pl.BlockSpec((_L * _HID, 16), lambda i: (0, 0)),
        pl.BlockSpec((1, 16), lambda i: (0, 0)),
    ],
    out_specs=[
        pl.BlockSpec((_BE, 16), lambda i: (i, 0)),
        pl.BlockSpec((16, _BE), lambda i: (0, i)),
    ],
    out_shape=[
        jax.ShapeDtypeStruct((_EPAD, 16), jnp.float32),
        jax.ShapeDtypeStruct((16, _EPAD), jnp.float32),
    ],
)


def _dinv_body(parts_ref, dinv_ref):
    deg = 1.0 + parts_ref[0] + parts_ref[1]
    dinv_ref[...] = lax.rsqrt(deg)


_dinv_call = pl.pallas_call(
    _dinv_body,
    out_shape=jax.ShapeDtypeStruct((_NP, 16), jnp.float32),
)


def _mm_scale_body(h_ref, wt_ref, dinv_ref, y_ref):
    xp = jnp.dot(h_ref[...], wt_ref[...], preferred_element_type=jnp.float32)
    y_ref[...] = dinv_ref[...] * xp


_mm_scale = pl.pallas_call(
    _mm_scale_body,
    out_shape=jax.ShapeDtypeStruct((_N, _D), jnp.float32),
)


def _mid_body(acc_ref, y_ref, dinv_ref, dinvn_ref, b_ref, a_ref,
              nw_ref, nb_ref, ms_ref, wtn_ref, ynext_ref, out_ref):
    acc = jnp.concatenate([acc_ref[0, :_N], acc_ref[1, :_N]], axis=1)
    out = dinv_ref[...] * (acc + y_ref[...]) + b_ref[...]
    a = a_ref[0, 0]
    t = jnp.where(out >= 0.0, out, a * out)
    mean = jnp.mean(t, axis=0, keepdims=True)
    o = t - mean * ms_ref[...]
    var = jnp.mean(o * o, axis=0, keepdims=True)
    g = nw_ref[...] * o * lax.rsqrt(var + 1e-5) + nb_ref[...]
    xp = jnp.dot(g, wtn_ref[...], preferred_element_type=jnp.float32)
    ynext_ref[...] = dinvn_ref[...] * xp
    out_ref[...] = out


_mid_call = pl.pallas_call(
    _mid_body,
    out_shape=[
        jax.ShapeDtypeStruct((_N, _D), jnp.float32),
        jax.ShapeDtypeStruct((_N, _D), jnp.float32),
    ],
)


_MESH = plsc.VectorSubcoreMesh(core_axis_name="c", subcore_axis_name="s")


def _zero_fill(ref, rows, cols16):
    z = jnp.zeros((16,), jnp.float32)

    def body(r, carry):
        for j in range(cols16):
            ref[r, pl.ds(16 * j, 16)] = z
        return carry

    lax.fori_loop(0, rows, body, 0)


@functools.partial(
    pl.kernel,
    out_type=jax.ShapeDtypeStruct((_NC, _NP, 16), jnp.float32),
    mesh=_MESH,
    scratch_types=[
        pltpu.VMEM((_CHUNK,), jnp.int32),
        pltpu.VMEM((_CHUNK, 16), jnp.float32),
        pltpu.VMEM((_RPS, 16), jnp.float32),
        pltpu.VMEM_SHARED((_NP, 16), jnp.float32),
        pltpu.SemaphoreType.DMA,
    ],
)
def _deg_kernel(col_hbm, wr_hbm, out_hbm, idx_v, w_v, buf_v, acc_sh, sem):
    cid = lax.axis_index("c")
    sid = lax.axis_index("s")
    wid = cid * _NS + sid

    _zero_fill(w_v, _CHUNK, 1)

    def _ramp(base):
        for g in range(_CHUNK // 16):
            idx_v[pl.ds(16 * g, 16)] = lax.iota(jnp.int32, 16) + (base + 16 * g)

    def zchunk(c, carry):
        _ramp(sid * _RPS + c * _CHUNK)
        pltpu.sync_copy(w_v, acc_sh.at[idx_v])
        return carry

    lax.fori_loop(0, _RPS // _CHUNK, zchunk, 0)
    plsc.subcore_barrier()

    def chunk(k, carry):
        base = wid * _EPW + k * _CHUNK
        pltpu.sync_copy(col_hbm.at[pl.ds(base, _CHUNK)], idx_v)
        pltpu.sync_copy(wr_hbm.at[pl.ds(base, _CHUNK)], w_v)
        pltpu.sync_copy(w_v, acc_sh.at[idx_v], add=True)
        return carry

    lax.fori_loop(0, _NCHUNK, chunk, 0)
    plsc.subcore_barrier()

    def cchunk(c, carry):
        base = sid * _RPS + c * _CHUNK
        _ramp(base)
        pltpu.sync_copy(acc_sh.at[idx_v], w_v)
        pltpu.sync_copy(w_v, out_hbm.at[cid, pl.ds(base, _CHUNK)])
        return carry

    lax.fori_loop(0, _RPS // _CHUNK, cchunk, 0)


@functools.partial(
    pl.kernel,
    out_type=jax.ShapeDtypeStruct((_NC, _NP, _DH), jnp.float32),
    mesh=_MESH,
    scratch_types=[
        pltpu.VMEM((_CHUNK,), jnp.int32),
        pltpu.VMEM((_CHUNK,), jnp.int32),
        pltpu.VMEM((_CHUNK,), jnp.int32),
        pltpu.VMEM((_CHUNK,), jnp.int32),
        pltpu.VMEM((_CHUNK,), jnp.float32),
        pltpu.VMEM((_CHUNK,), jnp.float32),
        pltpu.VMEM((_CHUNK, _D), jnp.float32),
        pltpu.VMEM((_CHUNK, _D), jnp.float32),
        pltpu.VMEM((_CHUNK, _DH), jnp.float32),
        pltpu.VMEM((_CHUNK, _DH), jnp.float32),
        pltpu.VMEM((_CHUNK,), jnp.int32),
        pltpu.VMEM((_CHUNK,), jnp.int32),
        pltpu.VMEM((64,), jnp.int32),
        pltpu.VMEM((64, _DH), jnp.float32),
        pltpu.VMEM_SHARED((_NHD, _DH), jnp.float32),
        pltpu.SemaphoreType.DMA,
        pltpu.SemaphoreType.DMA,
        pltpu.SemaphoreType.DMA,
        pltpu.SemaphoreType.DMA,
    ],
)
def _agg_kernel(y_hbm, row_hbm, col_hbm, wl_hbm, out_hbm,
                idx_r0, idx_r1, idx_c0, idx_c1, w_v0, w_v1,
                msg_v0, msg_v1, half_v0, half_v1, sidx0, sidx1,
                zidx_v, zbuf_v, acc_sh,
                gsem0, gsem1, ssem0, ssem1):
    cid = lax.axis_index("c")
    sid = lax.axis_index("s")
    foff = cid * _DH

    idx_r = (idx_r0, idx_r1)
    idx_c = (idx_c0, idx_c1)
    w_v = (w_v0, w_v1)
    msg_v = (msg_v0, msg_v1)
    half_v = (half_v0, half_v1)
    gsem = (gsem0, gsem1)
    ssem = (ssem0, ssem1)
    sidx = (sidx0, sidx1)

    _zero_fill(zbuf_v, 64, _DH // 16)

    def _ramp64(base):
        for g in range(4):
            zidx_v[pl.ds(16 * g, 16)] = lax.iota(jnp.int32, 16) + (base + 16 * g)

    def _prefetch(k, b, p):
        base = sid * _EPT + k * _CHUNK
        pltpu.sync_copy(row_hbm.at[pl.ds(base, _CHUNK)], idx_r[b])
        pltpu.async_copy(y_hbm.at[idx_r[b]], msg_v[b], gsem[b])
        pltpu.sync_copy(wl_hbm.at[pl.ds(base, _CHUNK)], w_v[b])
        pltpu.sync_copy(col_hbm.at[pl.ds(base, _CHUNK)], idx_c[b])
        for g in range(_CHUNK // 16):
            sl = pl.ds(16 * g, 16)
            lv = idx_c[b][sl] - (p * _NH)
            ok = (lv >= 0) & (lv < _NH)
            idx_c[b][sl] = jnp.where(ok, lv, _NH)

    def _wait_gather(b):
        pltpu.make_async_copy(y_hbm.at[idx_r[b]], msg_v[b], gsem[b]).wait()

    def _wait_scatter(b):
        pltpu.make_async_copy(half_v[b], acc_sh.at[sidx[b]], ssem[b]).wait()

    def _issue_scatter(b):
        for g in range(_CHUNK // 16):
            sl = pl.ds(16 * g, 16)
            sidx[b][sl] = idx_c[b][sl]
        pltpu.async_copy(half_v[b], acc_sh.at[sidx[b]], ssem[b], add=True)

    def _scale(b):
        def scale(g, c2):
            wvec = w_v[b][pl.ds(16 * g, 16)]
            for t in range(16):
                wv = wvec[t]
                i = 16 * g + t
                for j in range(_DH // 16):
                    v = msg_v[b][i, pl.ds(foff + 16 * j, 16)]
                    half_v[b][i, pl.ds(16 * j, 16)] = v * wv
            return c2

        lax.fori_loop(0, _CHUNK // 16, scale, 0)

    for p in range(2):
        def zchunk(c, carry):
            _ramp64(sid * _RPH + c * 64)
            pltpu.sync_copy(zbuf_v, acc_sh.at[zidx_v])
            return carry

        lax.fori_loop(0, _RPH // 64, zchunk, 0)
        plsc.subcore_barrier()

        _prefetch(0, 0, p)

        def pair(i, carry):
            _prefetch(2 * i + 1, 1, p)
            _wait_gather(0)
            pl.when(i >= 1)(lambda: _wait_scatter(0))
            _scale(0)
            _issue_scatter(0)
            pl.when(i < _NCHUNK2 // 2 - 1)(lambda: _prefetch(2 * i + 2, 0, p))
            _wait_gather(1)
            pl.when(i >= 1)(lambda: _wait_scatter(1))
            _scale(1)
            _issue_scatter(1)
            return carry

        lax.fori_loop(0, _NCHUNK2 // 2, pair, 0)
        _wait_scatter(0)
        _wait_scatter(1)
        plsc.subcore_barrier()

        def cchunk(c, carry):
            base = sid * _RPH + c * 64
            _ramp64(base)
            pltpu.sync_copy(acc_sh.at[zidx_v], zbuf_v)
            pltpu.sync_copy(zbuf_v, out_hbm.at[cid, pl.ds(p * _NH + base, 64)])
            return carry

        lax.fori_loop(0, _RPH // 64, cchunk, 0)
        _zero_fill(zbuf_v, 64, _DH // 16)
        plsc.subcore_barrier()


def kernel(x, edge_index, edge_attr, params):
    convs = params["convs"]
    row = edge_index[0]
    col = edge_index[1]
    pad = _EPAD - _E
    row_p = jnp.pad(row, (0, pad))
    col_p = jnp.pad(col, (0, pad))
    ea_p = jnp.pad(edge_attr, ((0, pad), (0, 0)))

    w1cat = jnp.concatenate([c["mW1"].T for c in convs], axis=1)
    b1cat = jnp.concatenate([c["mb1"] for c in convs])[None, :]
    w2blk = jnp.zeros((_L * _HID, 16), jnp.float32)
    b2cat = jnp.zeros((16,), jnp.float32)
    for l in range(_L):
        w2blk = w2blk.at[_HID * l:_HID * (l + 1), l].set(convs[l]["mW2"][0])
        b2cat = b2cat.at[l].set(convs[l]["mb2"][0])
    b2cat = b2cat[None, :]

    w_rows, w_t = _edge_mlp(ea_p, w1cat, b1cat, w2blk, b2cat)

    deg_parts = _deg_kernel(col_p, w_rows)
    dinv_all = _dinv_call(deg_parts)[:_N]

    wts = [c["W"].T for c in convs]
    dinvs = [lax.slice(dinv_all, (0, l), (_N, l + 1)) for l in range(_L)]

    y = _mm_scale(x, wts[0], dinvs[0])

    nrm = params["norms"]
    acts = params["acts"]
    xs = {
        "wl": w_t[:_L],
        "dinv": jnp.stack(dinvs),
        "dinvn": jnp.stack(dinvs[1:] + [dinvs[-1]]),
        "b": jnp.stack([c["b"][None, :] for c in convs]),
        "a": jnp.stack([a.reshape(1, 1) for a in acts] + [acts[0].reshape(1, 1)]),
        "nw": jnp.stack([n["weight"][None, :] for n in nrm] + [nrm[0]["weight"][None, :]]),
        "nb": jnp.stack([n["bias"][None, :] for n in nrm] + [nrm[0]["bias"][None, :]]),
        "ms": jnp.stack([n["mean_scale"][None, :] for n in nrm] + [nrm[0]["mean_scale"][None, :]]),
        "wtn": jnp.stack(wts[1:] + [wts[-1]]),
    }

    def step(carry, s):
        y_c, _ = carry
        acc = _agg_kernel(y_c, row_p, col_p, s["wl"])
        y_n, out = _mid_call(acc, y_c, s["dinv"], s["dinvn"], s["b"], s["a"],
                             s["nw"], s["nb"], s["ms"], s["wtn"])
        return (y_n, out), None

    (_, out), _ = lax.scan(step, (y, jnp.zeros((_N, _D), jnp.float32)), xs)
    return out


# speedup vs baseline: 4.5507x; 1.2256x over previous
import functools

import jax
import jax.numpy as jnp
from jax import lax
from jax.experimental import pallas as pl
from jax.experimental.pallas import tpu as pltpu
from jax.experimental.pallas import tpu_sc as plsc

_N = 10000
_E = 320000
_D = 128
_DH = _D // 2
_EDGE_DIM = 16
_HID = 32
_L = 10

_NC = 2
_NS = 16
_NW = _NC * _NS
_CHUNK = 128
_EPW = 10112
_NCHUNK = _EPW // _CHUNK
_EPAD = _NW * _EPW
_EPT = _EPAD // _NS
_NCHUNK2 = _EPT // _CHUNK
_BE = 4096
_NP = 10240
_RPS = _NP // _NS
_NH = 5120
_NHD = _NH + 8
_RPH = _NH // _NS


def _edge_mlp_body(ea_ref, w1_ref, b1_ref, w2_ref, b2_ref, wr_ref, wt_ref):
    i = pl.program_id(0)
    h = jnp.dot(ea_ref[...], w1_ref[...], preferred_element_type=jnp.float32)
    h = jnp.maximum(h + b1_ref[...], 0.0)
    z = jnp.dot(h, w2_ref[...], preferred_element_type=jnp.float32) + b2_ref[...]
    w = jax.nn.sigmoid(z)
    eid = i * _BE + lax.broadcasted_iota(jnp.int32, (_BE, _EDGE_DIM), 0)
    w = jnp.where(eid < _E, w, 0.0)
    wr_ref[...] = w
    wt_ref[...] = w.T


_edge_mlp = pl.pallas_call(
    _edge_mlp_body,
    grid=(_EPAD // _BE,),
    in_specs=[
        pl.BlockSpec((_BE, _EDGE_DIM), lambda i: (i, 0)),
        pl.BlockSpec((_EDGE_DIM, _L * _HID), lambda i: (0, 0)),
        pl.BlockSpec((1, _L * _HID), lambda i: (0, 0)),
        pl.BlockSpec((_L * _HID, 16), lambda i: (0, 0)),
        pl.BlockSpec((1, 16), lambda i: (0, 0)),
    ],
    out_specs=[
        pl.BlockSpec((_BE, 16), lambda i: (i, 0)),
        pl.BlockSpec((16, _BE), lambda i: (0, i)),
    ],
    out_shape=[
        jax.ShapeDtypeStruct((_EPAD, 16), jnp.float32),
        jax.ShapeDtypeStruct((16, _EPAD), jnp.float32),
    ],
)


def _dinv_body(parts_ref, dinv_ref):
    deg = 1.0 + parts_ref[0] + parts_ref[1]
    dinv_ref[...] = lax.rsqrt(deg)


_dinv_call = pl.pallas_call(
    _dinv_body,
    out_shape=jax.ShapeDtypeStruct((_NP, 16), jnp.float32),
)


def _mm_scale_body(h_ref, wt_ref, dinv_ref, y_ref):
    xp = jnp.dot(h_ref[...], wt_ref[...], preferred_element_type=jnp.float32)
    y_ref[...] = dinv_ref[...] * xp


_mm_scale = pl.pallas_call(
    _mm_scale_body,
    out_shape=jax.ShapeDtypeStruct((_N, _D), jnp.float32),
)


def _mid_body(acc_ref, y_ref, dinv_ref, dinvn_ref, b_ref, a_ref,
              nw_ref, nb_ref, ms_ref, wtn_ref, ynext_ref, out_ref):
    acc = jnp.concatenate([acc_ref[0, :_N], acc_ref[1, :_N]], axis=1)
    out = dinv_ref[...] * (acc + y_ref[...]) + b_ref[...]
    a = a_ref[0, 0]
    t = jnp.where(out >= 0.0, out, a * out)
    mean = jnp.mean(t, axis=0, keepdims=True)
    o = t - mean * ms_ref[...]
    var = jnp.mean(o * o, axis=0, keepdims=True)
    g = nw_ref[...] * o * lax.rsqrt(var + 1e-5) + nb_ref[...]
    xp = jnp.dot(g, wtn_ref[...], preferred_element_type=jnp.float32)
    ynext_ref[...] = dinvn_ref[...] * xp
    out_ref[...] = out


_mid_call = pl.pallas_call(
    _mid_body,
    out_shape=[
        jax.ShapeDtypeStruct((_N, _D), jnp.float32),
        jax.ShapeDtypeStruct((_N, _D), jnp.float32),
    ],
)


_MESH = plsc.VectorSubcoreMesh(core_axis_name="c", subcore_axis_name="s")


def _zero_fill(ref, rows, cols16):
    z = jnp.zeros((16,), jnp.float32)

    def body(r, carry):
        for j in range(cols16):
            ref[r, pl.ds(16 * j, 16)] = z
        return carry

    lax.fori_loop(0, rows, body, 0)


@functools.partial(
    pl.kernel,
    out_type=jax.ShapeDtypeStruct((_NC, _NP, 16), jnp.float32),
    mesh=_MESH,
    scratch_types=[
        pltpu.VMEM((_CHUNK,), jnp.int32),
        pltpu.VMEM((_CHUNK, 16), jnp.float32),
        pltpu.VMEM((_RPS, 16), jnp.float32),
        pltpu.VMEM_SHARED((_NP, 16), jnp.float32),
        pltpu.SemaphoreType.DMA,
    ],
)
def _deg_kernel(col_hbm, wr_hbm, out_hbm, idx_v, w_v, buf_v, acc_sh, sem):
    cid = lax.axis_index("c")
    sid = lax.axis_index("s")
    wid = cid * _NS + sid

    _zero_fill(w_v, _CHUNK, 1)

    def _ramp(base):
        for g in range(_CHUNK // 16):
            idx_v[pl.ds(16 * g, 16)] = lax.iota(jnp.int32, 16) + (base + 16 * g)

    def zchunk(c, carry):
        _ramp(sid * _RPS + c * _CHUNK)
        pltpu.sync_copy(w_v, acc_sh.at[idx_v])
        return carry

    lax.fori_loop(0, _RPS // _CHUNK, zchunk, 0)
    plsc.subcore_barrier()

    def chunk(k, carry):
        base = wid * _EPW + k * _CHUNK
        pltpu.sync_copy(col_hbm.at[pl.ds(base, _CHUNK)], idx_v)
        pltpu.sync_copy(wr_hbm.at[pl.ds(base, _CHUNK)], w_v)
        pltpu.sync_copy(w_v, acc_sh.at[idx_v], add=True)
        return carry

    lax.fori_loop(0, _NCHUNK, chunk, 0)
    plsc.subcore_barrier()

    def cchunk(c, carry):
        base = sid * _RPS + c * _CHUNK
        _ramp(base)
        pltpu.sync_copy(acc_sh.at[idx_v], w_v)
        pltpu.sync_copy(w_v, out_hbm.at[cid, pl.ds(base, _CHUNK)])
        return carry

    lax.fori_loop(0, _RPS // _CHUNK, cchunk, 0)


@functools.partial(
    pl.kernel,
    out_type=jax.ShapeDtypeStruct((_NC, _NP, _DH), jnp.float32),
    mesh=_MESH,
    scratch_types=[
        pltpu.VMEM((_CHUNK,), jnp.int32),
        pltpu.VMEM((_CHUNK,), jnp.int32),
        pltpu.VMEM((_CHUNK,), jnp.int32),
        pltpu.VMEM((_CHUNK,), jnp.int32),
        pltpu.VMEM((_CHUNK,), jnp.float32),
        pltpu.VMEM((_CHUNK,), jnp.float32),
        pltpu.VMEM((_CHUNK, _D), jnp.float32),
        pltpu.VMEM((_CHUNK, _D), jnp.float32),
        pltpu.VMEM((_CHUNK, _DH), jnp.float32),
        pltpu.VMEM((_CHUNK, _DH), jnp.float32),
        pltpu.VMEM((_CHUNK,), jnp.int32),
        pltpu.VMEM((_CHUNK,), jnp.int32),
        pltpu.VMEM((64,), jnp.int32),
        pltpu.VMEM((64, _DH), jnp.float32),
        pltpu.VMEM_SHARED((_NHD, _DH), jnp.float32),
        pltpu.SemaphoreType.DMA,
        pltpu.SemaphoreType.DMA,
        pltpu.SemaphoreType.DMA,
        pltpu.SemaphoreType.DMA,
    ],
)
def _agg_kernel(y_hbm, row_hbm, col_hbm, wl_hbm, out_hbm,
                idx_r0, idx_r1, idx_c0, idx_c1, w_v0, w_v1,
                msg_v0, msg_v1, half_v0, half_v1, sidx0, sidx1,
                zidx_v, zbuf_v, acc_sh,
                gsem0, gsem1, ssem0, ssem1):
    cid = lax.axis_index("c")
    sid = lax.axis_index("s")
    foff = cid * _DH

    idx_r = (idx_r0, idx_r1)
    idx_c = (idx_c0, idx_c1)
    w_v = (w_v0, w_v1)
    msg_v = (msg_v0, msg_v1)
    half_v = (half_v0, half_v1)
    gsem = (gsem0, gsem1)
    ssem = (ssem0, ssem1)
    sidx = (sidx0, sidx1)

    _zero_fill(zbuf_v, 64, _DH // 16)

    def _ramp64(base):
        for g in range(4):
            zidx_v[pl.ds(16 * g, 16)] = lax.iota(jnp.int32, 16) + (base + 16 * g)

    def _prefetch(k, b, p):
        base = sid * _EPT + k * _CHUNK
        pltpu.sync_copy(row_hbm.at[pl.ds(base, _CHUNK)], idx_r[b])
        pltpu.async_copy(y_hbm.at[idx_r[b]], msg_v[b], gsem[b])
        pltpu.sync_copy(wl_hbm.at[pl.ds(base, _CHUNK)], w_v[b])
        pltpu.sync_copy(col_hbm.at[pl.ds(base, _CHUNK)], idx_c[b])
        for g in range(_CHUNK // 16):
            sl = pl.ds(16 * g, 16)
            lv = idx_c[b][sl] - (p * _NH)
            ok = (lv >= 0) & (lv < _NH)
            idx_c[b][sl] = jnp.where(ok, lv, _NH)

    def _wait_gather(b):
        pltpu.make_async_copy(y_hbm.at[idx_r[b]], msg_v[b], gsem[b]).wait()

    def _wait_scatter(b):
        pltpu.make_async_copy(half_v[b], acc_sh.at[sidx[b]], ssem[b]).wait()

    def _issue_scatter(b):
        for g in range(_CHUNK // 16):
            sl = pl.ds(16 * g, 16)
            sidx[b][sl] = idx_c[b][sl]
        pltpu.async_copy(half_v[b], acc_sh.at[sidx[b]], ssem[b], add=True)

    def _scale(b):
        def scale(g):
            wvec = w_v[b][pl.ds(16 * g, 16)]
            for t in range(16):
                wv = wvec[t]
                i = 16 * g + t
                for j in range(_DH // 16):
                    v = msg_v[b][i, pl.ds(foff + 16 * j, 16)]
                    half_v[b][i, pl.ds(16 * j, 16)] = v * wv

        plsc.parallel_loop(0, _CHUNK // 16, 1, unroll=2)(scale)

    for p in range(2):
        def zchunk(c, carry):
            _ramp64(sid * _RPH + c * 64)
            pltpu.sync_copy(zbuf_v, acc_sh.at[zidx_v])
            return carry

        lax.fori_loop(0, _RPH // 64, zchunk, 0)
        plsc.subcore_barrier()

        _prefetch(0, 0, p)

        def pair(i, carry):
            _prefetch(2 * i + 1, 1, p)
            _wait_gather(0)
            pl.when(i >= 1)(lambda: _wait_scatter(0))
            _scale(0)
            _issue_scatter(0)
            pl.when(i < _NCHUNK2 // 2 - 1)(lambda: _prefetch(2 * i + 2, 0, p))
            _wait_gather(1)
            pl.when(i >= 1)(lambda: _wait_scatter(1))
            _scale(1)
            _issue_scatter(1)
            return carry

        lax.fori_loop(0, _NCHUNK2 // 2, pair, 0)
        _wait_scatter(0)
        _wait_scatter(1)
        plsc.subcore_barrier()

        def cchunk(c, carry):
            base = sid * _RPH + c * 64
            _ramp64(base)
            pltpu.sync_copy(acc_sh.at[zidx_v], zbuf_v)
            pltpu.sync_copy(zbuf_v, out_hbm.at[cid, pl.ds(p * _NH + base, 64)])
            return carry

        lax.fori_loop(0, _RPH // 64, cchunk, 0)
        _zero_fill(zbuf_v, 64, _DH // 16)
        plsc.subcore_barrier()


def kernel(x, edge_index, edge_attr, params):
    convs = params["convs"]
    row = edge_index[0]
    col = edge_index[1]
    pad = _EPAD - _E
    row_p = jnp.pad(row, (0, pad))
    col_p = jnp.pad(col, (0, pad))
    ea_p = jnp.pad(edge_attr, ((0, pad), (0, 0)))

    w1cat = jnp.concatenate([c["mW1"].T for c in convs], axis=1)
    b1cat = jnp.concatenate([c["mb1"] for c in convs])[None, :]
    w2blk = jnp.zeros((_L * _HID, 16), jnp.float32)
    b2cat = jnp.zeros((16,), jnp.float32)
    for l in range(_L):
        w2blk = w2blk.at[_HID * l:_HID * (l + 1), l].set(convs[l]["mW2"][0])
        b2cat = b2cat.at[l].set(convs[l]["mb2"][0])
    b2cat = b2cat[None, :]

    w_rows, w_t = _edge_mlp(ea_p, w1cat, b1cat, w2blk, b2cat)

    deg_parts = _deg_kernel(col_p, w_rows)
    dinv_all = _dinv_call(deg_parts)[:_N]

    wts = [c["W"].T for c in convs]
    dinvs = [lax.slice(dinv_all, (0, l), (_N, l + 1)) for l in range(_L)]

    y = _mm_scale(x, wts[0], dinvs[0])

    nrm = params["norms"]
    acts = params["acts"]
    xs = {
        "wl": w_t[:_L],
        "dinv": jnp.stack(dinvs),
        "dinvn": jnp.stack(dinvs[1:] + [dinvs[-1]]),
        "b": jnp.stack([c["b"][None, :] for c in convs]),
        "a": jnp.stack([a.reshape(1, 1) for a in acts] + [acts[0].reshape(1, 1)]),
        "nw": jnp.stack([n["weight"][None, :] for n in nrm] + [nrm[0]["weight"][None, :]]),
        "nb": jnp.stack([n["bias"][None, :] for n in nrm] + [nrm[0]["bias"][None, :]]),
        "ms": jnp.stack([n["mean_scale"][None, :] for n in nrm] + [nrm[0]["mean_scale"][None, :]]),
        "wtn": jnp.stack(wts[1:] + [wts[-1]]),
    }

    def step(carry, s):
        y_c, _ = carry
        acc = _agg_kernel(y_c, row_p, col_p, s["wl"])
        y_n, out = _mid_call(acc, y_c, s["dinv"], s["dinvn"], s["b"], s["a"],
                             s["nw"], s["nb"], s["ms"], s["wtn"])
        return (y_n, out), None

    (_, out), _ = lax.scan(step, (y, jnp.zeros((_N, _D), jnp.float32)), xs)
    return out
